# Initial kernel scaffold; baseline (speedup 1.0000x reference)
import numpy as np

import jax
import jax.numpy as jnp
from jax.experimental import pallas as pl
from jax.experimental.pallas import tpu as pltpu

NEG = 0.2
_VMEM_LIMIT = 100 * 1024 * 1024


def _lrelu(y):
    return jnp.maximum(y, NEG * y)


def _tap_lhs(src, dy, ho, wo, stride, batched=False):
    if stride == 1:
        sl = lambda dx: (slice(dy, dy + ho), slice(dx, dx + wo))
    else:
        sl = lambda dx: (pl.ds(dy, ho, 2), pl.ds(dx, wo, 2))
    if batched:
        parts = [src[:, sl(dx)[0], sl(dx)[1], :] for dx in range(3)]
    else:
        parts = [src[sl(dx)[0], sl(dx)[1], :] for dx in range(3)]
    lhs = jnp.concatenate(parts, axis=-1)
    m = lhs.shape[0] * ho * wo if batched else ho * wo
    return lhs.reshape(m, lhs.shape[-1])


def _conv_layer(src, w_ref, s_ref, h_ref, hi, stride, batched=False):
    ho = (hi - 3) // stride + 1
    cin = src.shape[-1]
    acc = None
    for dy in range(3):
        lhs = _tap_lhs(src, dy, ho, ho, stride, batched)
        wk = w_ref[dy * 3 * cin:(dy + 1) * 3 * cin, :]
        d = jnp.dot(lhs, wk, preferred_element_type=jnp.float32)
        acc = d if acc is None else acc + d
    y = acc * s_ref[...] + h_ref[...]
    return _lrelu(y).astype(jnp.bfloat16), ho


def _convA_body(x0_ref, w0, s0, h0, w1, s1, h1, w2, s2, h2, w3, s3, h3,
                w4, s4, h4, out_ref, a0, a1, a2, a3):
    y = jnp.dot(x0_ref[0], w0[...], preferred_element_type=jnp.float32)
    y = _lrelu(y * s0[...] + h0[...])
    a0[...] = y.reshape(126, 126, 64).astype(jnp.bfloat16)
    y, ho = _conv_layer(a0, w1, s1, h1, 126, 2)
    a1[...] = y.reshape(ho, ho, 64)
    y, ho = _conv_layer(a1, w2, s2, h2, 62, 1)
    a2[...] = y.reshape(ho, ho, 128)
    y, ho = _conv_layer(a2, w3, s3, h3, 60, 2)
    a3[...] = y.reshape(ho, ho, 128)
    y, ho = _conv_layer(a3, w4, s4, h4, 29, 1)
    out_ref[0] = y.reshape(ho, ho, 256)


def _convB_body(a4_ref, w5, s5, h5, w6, s6, h6, w7, s7, h7, p8_ref,
                out_ref, a5, a6):
    nb = a4_ref.shape[0]
    y, ho = _conv_layer(a4_ref, w5, s5, h5, 27, 2, batched=True)
    a5[...] = y.reshape(nb, ho, ho, 256)
    y, ho = _conv_layer(a5, w6, s6, h6, 13, 1, batched=True)
    a6[...] = y.reshape(nb, ho, ho, 512)
    y, _ = _conv_layer(a6, w7, s7, h7, 11, 2, batched=True)
    pooled = jnp.dot(p8_ref[...], y.astype(jnp.float32),
                     preferred_element_type=jnp.float32)
    out_ref[...] = pooled.reshape(nb, 36, 512).astype(jnp.bfloat16)


def _head_body(f_ref, w1_ref, b1_ref, w2_ref, b2_ref, o_ref, hacc):
    k = pl.program_id(1)

    @pl.when(k == 0)
    def _():
        hacc[...] = jnp.zeros_like(hacc)

    hacc[...] += jnp.dot(f_ref[...], w1_ref[...],
                         preferred_element_type=jnp.float32)

    @pl.when(k == pl.num_programs(1) - 1)
    def _():
        h = _lrelu(hacc[...] + b1_ref[...])
        o = jnp.sum(h * w2_ref[...], axis=-1, keepdims=True)
        o_ref[...] = o + jnp.where(pl.program_id(0) == 0, b2_ref[0, 0], 0.0)


def _pool_matrix(n_in, n_out):
    p = np.zeros((n_out, n_in), np.float32)
    for o in range(n_out):
        s = (o * n_in) // n_out
        e = -(-((o + 1) * n_in) // n_out)
        p[o, s:e] = 1.0 / (e - s)
    return p


_P8 = np.kron(np.eye(8, dtype=np.float32),
              np.kron(_pool_matrix(5, 6), _pool_matrix(5, 6)))


def kernel(x, input_w, input_scale, input_shift,
           block0_w, block0_scale, block0_shift,
           block1_w, block1_scale, block1_shift,
           block2_w, block2_scale, block2_shift,
           block3_w, block3_scale, block3_shift,
           block4_w, block4_scale, block4_shift,
           block5_w, block5_scale, block5_shift,
           block6_w, block6_scale, block6_shift,
           fc1_w, fc1_b, fc2_w, fc2_b):
    n = x.shape[0]
    xh = jnp.transpose(x, (0, 2, 3, 1)).astype(jnp.bfloat16)
    taps = [xh[:, dy:dy + 126, dx:dx + 126, :]
            for dy in range(3) for dx in range(3)]
    x0 = jnp.stack(taps, axis=3).reshape(n, 126 * 126, 27)

    wspec = lambda shp: pl.BlockSpec(shp, lambda i: tuple(0 for _ in shp))
    f32 = jnp.float32

    a_params = []
    a_specs = [pl.BlockSpec((1, 126 * 126, 27), lambda i: (i, 0, 0))]
    for w, s, h in ((input_w, input_scale, input_shift),
                    (block0_w, block0_scale, block0_shift),
                    (block1_w, block1_scale, block1_shift),
                    (block2_w, block2_scale, block2_shift),
                    (block3_w, block3_scale, block3_shift)):
        a_params += [w, s, h]
        a_specs += [wspec(w.shape), wspec(s.shape), wspec(h.shape)]
    a4 = pl.pallas_call(
        _convA_body,
        out_shape=jax.ShapeDtypeStruct((n, 27, 27, 256), jnp.bfloat16),
        grid=(n,),
        in_specs=a_specs,
        out_specs=pl.BlockSpec((1, 27, 27, 256), lambda i: (i, 0, 0, 0)),
        scratch_shapes=[
            pltpu.VMEM((126, 126, 64), jnp.bfloat16),
            pltpu.VMEM((62, 62, 64), jnp.bfloat16),
            pltpu.VMEM((60, 60, 128), jnp.bfloat16),
            pltpu.VMEM((29, 29, 128), jnp.bfloat16),
        ],
        compiler_params=pltpu.CompilerParams(
            dimension_semantics=("parallel",),
            vmem_limit_bytes=_VMEM_LIMIT,
        ),
    )(x0, *a_params)

    b_params = []
    b_specs = [pl.BlockSpec((8, 27, 27, 256), lambda i: (i, 0, 0, 0))]
    for w, s, h in ((block4_w, block4_scale, block4_shift),
                    (block5_w, block5_scale, block5_shift),
                    (block6_w, block6_scale, block6_shift)):
        b_params += [w, s, h]
        b_specs += [wspec(w.shape), wspec(s.shape), wspec(h.shape)]
    b_specs.append(wspec((288, 200)))
    pooled = pl.pallas_call(
        _convB_body,
        out_shape=jax.ShapeDtypeStruct((n, 36, 512), jnp.bfloat16),
        grid=(n // 8,),
        in_specs=b_specs,
        out_specs=pl.BlockSpec((8, 36, 512), lambda i: (i, 0, 0)),
        scratch_shapes=[
            pltpu.VMEM((8, 13, 13, 256), jnp.bfloat16),
            pltpu.VMEM((8, 11, 11, 512), jnp.bfloat16),
        ],
        compiler_params=pltpu.CompilerParams(
            dimension_semantics=("parallel",),
            vmem_limit_bytes=_VMEM_LIMIT,
        ),
    )(a4, *b_params, jnp.asarray(_P8))

    feat = pooled.reshape(n, 36 * 512)
    kf, n1 = fc1_w.shape
    tn, tk = n1 // 2, kf // 4
    partial = pl.pallas_call(
        _head_body,
        out_shape=jax.ShapeDtypeStruct((n, 2), f32),
        grid=(2, kf // tk),
        in_specs=[
            pl.BlockSpec((n, tk), lambda j, k: (0, k)),
            pl.BlockSpec((tk, tn), lambda j, k: (k, j)),
            pl.BlockSpec((1, tn), lambda j, k: (0, j)),
            pl.BlockSpec((1, tn), lambda j, k: (0, j)),
            pl.BlockSpec((1, 1), lambda j, k: (0, 0)),
        ],
        out_specs=pl.BlockSpec((n, 1), lambda j, k: (0, j)),
        scratch_shapes=[pltpu.VMEM((n, tn), f32)],
        compiler_params=pltpu.CompilerParams(
            dimension_semantics=("parallel", "arbitrary"),
            vmem_limit_bytes=_VMEM_LIMIT,
        ),
    )(feat, fc1_w, fc1_b.reshape(1, n1),
      fc2_w.reshape(1, n1).astype(f32), fc2_b.reshape(1, 1))
    return partial[:, :1] + partial[:, 1:]


# speedup vs baseline: 19.1408x; 19.1408x over previous
import numpy as np

import jax
import jax.numpy as jnp
from jax.experimental import pallas as pl
from jax.experimental.pallas import tpu as pltpu

NEG = 0.2
_VMEM_LIMIT = 100 * 1024 * 1024


def _lrelu(y):
    return jnp.maximum(y, NEG * y)


def _tap_lhs(src, dy, ho, wo, stride, batched=False):
    refs = src if isinstance(src, (list, tuple)) else [src]
    if stride == 1:
        sl = lambda dx: (slice(dy, dy + ho), slice(dx, dx + wo))
    else:
        sl = lambda dx: (pl.ds(dy, ho, 2), pl.ds(dx, wo, 2))
    if batched:
        parts = [r[:, sl(dx)[0], sl(dx)[1], :] for dx in range(3) for r in refs]
    else:
        parts = [r[sl(dx)[0], sl(dx)[1], :] for dx in range(3) for r in refs]
    lhs = jnp.concatenate(parts, axis=-1).astype(jnp.bfloat16)
    m = lhs.shape[0] * ho * wo if batched else ho * wo
    return lhs.reshape(m, lhs.shape[-1])


def _conv_layer(src, w_ref, s_ref, h_ref, hi, stride, batched=False):
    ho = (hi - 3) // stride + 1
    refs = src if isinstance(src, (list, tuple)) else [src]
    cin = sum(r.shape[-1] for r in refs)
    acc = None
    for dy in range(3):
        lhs = _tap_lhs(src, dy, ho, ho, stride, batched)
        wk = w_ref[dy * 3 * cin:(dy + 1) * 3 * cin, :]
        d = jnp.dot(lhs, wk, preferred_element_type=jnp.float32)
        acc = d if acc is None else acc + d
    y = acc * s_ref[...] + h_ref[...]
    return _lrelu(y).astype(jnp.bfloat16), ho


def _convA_body(x0_ref, w0, s0, h0, w1, s1, h1, w2, s2, h2, w3, s3, h3,
                w4, s4, h4, out_ref, a0, a1, a2, a3):
    y = jnp.dot(x0_ref[0], w0[...], preferred_element_type=jnp.float32)
    y = _lrelu(y * s0[...] + h0[...])
    a0[...] = y.reshape(126, 126, 64).astype(jnp.bfloat16).astype(a0.dtype)
    y, ho = _conv_layer(a0, w1, s1, h1, 126, 2)
    a1[...] = y.reshape(ho, ho, 64).astype(a1.dtype)
    y, ho = _conv_layer(a1, w2, s2, h2, 62, 1)
    a2[...] = y.reshape(ho, ho, 128).astype(a2.dtype)
    y, ho = _conv_layer(a2, w3, s3, h3, 60, 2)
    a3[...] = y.reshape(ho, ho, 128).astype(a3.dtype)
    y, ho = _conv_layer(a3, w4, s4, h4, 29, 1)
    out_ref[0] = y.reshape(ho, ho, 256)


def _convB_body(a4_ref, w5, s5, h5, w6, s6, h6, w7, s7, h7, p8_ref,
                out_ref, a4c0, a4c1, a50, a60, a61, a62, a63):
    nb = a4_ref.shape[0]
    a4c = [a4c0, a4c1]
    a6 = [a60, a61, a62, a63]
    for i, r in enumerate(a4c):
        r[...] = a4_ref[:, :, :, 128 * i:128 * (i + 1)].astype(r.dtype)
    y, ho = _conv_layer(a4c, w5, s5, h5, 27, 2, batched=True)
    a50[...] = y.reshape(nb, ho, ho, 256).astype(a50.dtype)
    y, ho = _conv_layer(a50, w6, s6, h6, 13, 1, batched=True)
    y = y.reshape(nb, ho, ho, 512)
    for i, r in enumerate(a6):
        r[...] = y[:, :, :, 128 * i:128 * (i + 1)].astype(r.dtype)
    y, _ = _conv_layer(a6, w7, s7, h7, 11, 2, batched=True)
    pooled = jnp.dot(p8_ref[...], y.astype(jnp.float32),
                     preferred_element_type=jnp.float32)
    out_ref[...] = pooled.reshape(nb, 36, 512).astype(jnp.bfloat16)


def _head_body(f_ref, w1_ref, b1_ref, w2_ref, b2_ref, o_ref, hacc):
    k = pl.program_id(1)

    @pl.when(k == 0)
    def _():
        hacc[...] = jnp.zeros_like(hacc)

    hacc[...] += jnp.dot(f_ref[...], w1_ref[...],
                         preferred_element_type=jnp.float32)

    @pl.when(k == pl.num_programs(1) - 1)
    def _():
        h = _lrelu(hacc[...] + b1_ref[...])
        o = jnp.sum(h * w2_ref[...], axis=-1, keepdims=True)
        o_ref[0] = o + jnp.where(pl.program_id(0) == 0, b2_ref[0, 0], 0.0)


def _pool_matrix(n_in, n_out):
    p = np.zeros((n_out, n_in), np.float32)
    for o in range(n_out):
        s = (o * n_in) // n_out
        e = -(-((o + 1) * n_in) // n_out)
        p[o, s:e] = 1.0 / (e - s)
    return p


_P8 = np.kron(np.eye(8, dtype=np.float32),
              np.kron(_pool_matrix(5, 6), _pool_matrix(5, 6)))


def kernel(x, input_w, input_scale, input_shift,
           block0_w, block0_scale, block0_shift,
           block1_w, block1_scale, block1_shift,
           block2_w, block2_scale, block2_shift,
           block3_w, block3_scale, block3_shift,
           block4_w, block4_scale, block4_shift,
           block5_w, block5_scale, block5_shift,
           block6_w, block6_scale, block6_shift,
           fc1_w, fc1_b, fc2_w, fc2_b):
    n = x.shape[0]
    xh = jnp.transpose(x, (0, 2, 3, 1)).astype(jnp.bfloat16)
    taps = [xh[:, dy:dy + 126, dx:dx + 126, :]
            for dy in range(3) for dx in range(3)]
    x0 = jnp.stack(taps, axis=3).reshape(n, 126 * 126, 27)

    wspec = lambda shp: pl.BlockSpec(shp, lambda i: tuple(0 for _ in shp))
    f32 = jnp.float32

    a_params = []
    a_specs = [pl.BlockSpec((1, 126 * 126, 27), lambda i: (i, 0, 0))]
    for w, s, h in ((input_w, input_scale, input_shift),
                    (block0_w, block0_scale, block0_shift),
                    (block1_w, block1_scale, block1_shift),
                    (block2_w, block2_scale, block2_shift),
                    (block3_w, block3_scale, block3_shift)):
        a_params += [w, s, h]
        a_specs += [wspec(w.shape), wspec(s.shape), wspec(h.shape)]
    a4 = pl.pallas_call(
        _convA_body,
        out_shape=jax.ShapeDtypeStruct((n, 27, 27, 256), jnp.bfloat16),
        grid=(n,),
        in_specs=a_specs,
        out_specs=pl.BlockSpec((1, 27, 27, 256), lambda i: (i, 0, 0, 0)),
        scratch_shapes=[
            pltpu.VMEM((126, 126, 64), jnp.float32),
            pltpu.VMEM((62, 62, 64), jnp.bfloat16),
            pltpu.VMEM((60, 60, 128), jnp.float32),
            pltpu.VMEM((29, 29, 128), jnp.bfloat16),
        ],
        compiler_params=pltpu.CompilerParams(
            dimension_semantics=("parallel",),
            vmem_limit_bytes=_VMEM_LIMIT,
        ),
    )(x0, *a_params)

    b_params = []
    b_specs = [pl.BlockSpec((8, 27, 27, 256), lambda i: (i, 0, 0, 0))]
    for w, s, h in ((block4_w, block4_scale, block4_shift),
                    (block5_w, block5_scale, block5_shift),
                    (block6_w, block6_scale, block6_shift)):
        b_params += [w, s, h]
        b_specs += [wspec(w.shape), wspec(s.shape), wspec(h.shape)]
    b_specs.append(wspec((288, 200)))
    pooled = pl.pallas_call(
        _convB_body,
        out_shape=jax.ShapeDtypeStruct((n, 36, 512), jnp.bfloat16),
        grid=(n // 8,),
        in_specs=b_specs,
        out_specs=pl.BlockSpec((8, 36, 512), lambda i: (i, 0, 0)),
        scratch_shapes=[
            pltpu.VMEM((8, 27, 27, 128), jnp.float32),
            pltpu.VMEM((8, 27, 27, 128), jnp.float32),
            pltpu.VMEM((8, 13, 13, 256), jnp.bfloat16),
            pltpu.VMEM((8, 11, 11, 128), jnp.float32),
            pltpu.VMEM((8, 11, 11, 128), jnp.float32),
            pltpu.VMEM((8, 11, 11, 128), jnp.float32),
            pltpu.VMEM((8, 11, 11, 128), jnp.float32),
        ],
        compiler_params=pltpu.CompilerParams(
            dimension_semantics=("parallel",),
            vmem_limit_bytes=_VMEM_LIMIT,
        ),
    )(a4, *b_params, jnp.asarray(_P8))

    feat = pooled.reshape(n, 36 * 512)
    kf, n1 = fc1_w.shape
    tn, tk = n1 // 2, kf // 4
    partial = pl.pallas_call(
        _head_body,
        out_shape=jax.ShapeDtypeStruct((2, n, 1), f32),
        grid=(2, kf // tk),
        in_specs=[
            pl.BlockSpec((n, tk), lambda j, k: (0, k)),
            pl.BlockSpec((tk, tn), lambda j, k: (k, j)),
            pl.BlockSpec((1, tn), lambda j, k: (0, j)),
            pl.BlockSpec((1, tn), lambda j, k: (0, j)),
            pl.BlockSpec((1, 1), lambda j, k: (0, 0)),
        ],
        out_specs=pl.BlockSpec((1, n, 1), lambda j, k: (j, 0, 0)),
        scratch_shapes=[pltpu.VMEM((n, tn), f32)],
        compiler_params=pltpu.CompilerParams(
            dimension_semantics=("parallel", "arbitrary"),
            vmem_limit_bytes=_VMEM_LIMIT,
        ),
    )(feat, fc1_w, fc1_b.reshape(1, n1),
      fc2_w.reshape(1, n1).astype(f32), fc2_b.reshape(1, 1))
    return partial[0] + partial[1]
